# baseline (speedup 1.0000x reference)
import functools

import jax
import jax.numpy as jnp
from jax import lax
from jax.experimental import pallas as pl
from jax.experimental.pallas import tpu as pltpu
from jax.experimental.pallas import tpu_sc as plsc

B, T, D = 16, 576, 256
KSIZE = 1024
NROW = B * T


BR = 3072
CH = 256


def _make_tc_body(br):
    def _tc_body(f_ref, k_ref, idx_ref, conf_ref):
        return _tc_top2_impl(f_ref, k_ref, idx_ref, conf_ref, br)
    return _tc_body


def _tc_top2_impl(f_ref, k_ref, idx_ref, conf_ref, BR):
    f = f_ref[...]
    big = jnp.float32(jnp.inf)

    m1 = jnp.full((BR,), big)
    i1 = jnp.full((BR,), KSIZE, jnp.int32)
    m2 = jnp.full((BR,), big)
    for j in range(KSIZE // CH):
        kc = k_ref[j * CH:(j + 1) * CH, :]
        s = lax.dot_general(
            kc, f, (((1,), (1,)), ((), ())),
            preferred_element_type=jnp.float32,
            precision=lax.Precision.DEFAULT,
        )
        knc = jnp.sum(kc * kc, axis=1, keepdims=True)
        s = knc - 2.0 * s
        cm1 = jnp.min(s, axis=0)
        col = lax.broadcasted_iota(jnp.int32, (CH, BR), 0) + j * CH
        ci1 = jnp.min(jnp.where(s == cm1[None, :], col, KSIZE), axis=0)
        s2 = jnp.where(col == ci1[None, :], big, s)
        cm2 = jnp.min(s2, axis=0)
        i1 = jnp.where(cm1 < m1, ci1, i1)
        m2 = jnp.minimum(jnp.maximum(m1, cm1), jnp.minimum(m2, cm2))
        m1 = jnp.minimum(m1, cm1)

    fn = jnp.sum(f * f, axis=1)
    d1 = jnp.sqrt(jnp.maximum(m1 + fn, 1e-12))
    d2 = jnp.sqrt(jnp.maximum(m2 + fn, 1e-12))
    idx_ref[...] = i1
    conf = 1.0 - d1 / (d2 + 1e-8)
    conf_ref[...] = jnp.broadcast_to(conf[:, None], (BR, 16))


def _tc_top2(f, K, br=BR, interpret=False):
    nrow = f.shape[0]
    grid = nrow // br
    return pl.pallas_call(
        _make_tc_body(br),
        grid=(grid,),
        in_specs=[
            pl.BlockSpec((br, D), lambda i: (i, 0)),
            pl.BlockSpec((KSIZE, D), lambda i: (0, 0)),
        ],
        out_specs=[
            pl.BlockSpec((br,), lambda i: (i,)),
            pl.BlockSpec((br, 16), lambda i: (i, 0)),
        ],
        out_shape=[
            jax.ShapeDtypeStruct((nrow,), jnp.int32),
            jax.ShapeDtypeStruct((nrow, 16), jnp.float32),
        ],
        interpret=interpret,
    )(f, K)


NC, NS, L = 2, 16, 16
NW = NC * NS
RPW = NROW // NW
DCH = D // L


def _make_sc_body(rpw):
    def _sc_body(k_hbm, idx_hbm, conf_hbm, out_hbm, idx_v, conf_v, rows_v,
                 sem):
        wid = lax.axis_index("s") * NC + lax.axis_index("c")
        base = wid * rpw
        pltpu.sync_copy(idx_hbm.at[pl.ds(base, rpw)], idx_v)
        pltpu.sync_copy(conf_hbm.at[pl.ds(base, rpw)], conf_v)
        pltpu.async_copy(k_hbm.at[idx_v], rows_v, sem).wait()

        @plsc.parallel_loop(0, rpw, 1, unroll=2)
        def _(r):
            cv = conf_v[r, :]
            for d in range(DCH):
                sl = pl.ds(d * L, L)
                rows_v[r, sl] = rows_v[r, sl] * cv

        pltpu.sync_copy(rows_v, out_hbm.at[pl.ds(base, rpw)])
    return _sc_body


@functools.cache
def _sc_gather_scale(nrow):
    rpw = nrow // NW
    return pl.kernel(
        _make_sc_body(rpw),
        out_type=jax.ShapeDtypeStruct((nrow, D), jnp.float32),
        mesh=plsc.VectorSubcoreMesh(core_axis_name="c", subcore_axis_name="s"),
        scratch_types=[
            pltpu.VMEM((rpw,), jnp.int32),
            pltpu.VMEM((rpw, L), jnp.float32),
            pltpu.VMEM((rpw, D), jnp.float32),
            pltpu.SemaphoreType.DMA,
        ],
    )


NSPLIT = 2
HROW = NROW // NSPLIT


def kernel(f_ipm, K):
    orig_shape = f_ipm.shape
    f = f_ipm.reshape(-1, orig_shape[-1])
    outs = []
    for h in range(NSPLIT):
        fh = lax.slice_in_dim(f, h * HROW, (h + 1) * HROW, axis=0)
        idx, conf = _tc_top2(fh, K, br=512)
        outs.append(_sc_gather_scale(HROW)(K, idx, conf))
    out = jnp.concatenate(outs, axis=0)
    return out.reshape(orig_shape)


# speedup vs baseline: 1.2854x; 1.2854x over previous
import functools

import jax
import jax.numpy as jnp
from jax import lax
from jax.experimental import pallas as pl
from jax.experimental.pallas import tpu as pltpu
from jax.experimental.pallas import tpu_sc as plsc

B, T, D = 16, 576, 256
KSIZE = 1024
NROW = B * T


BR = 3072
CH = 1024


def _make_tc_body(br):
    def _tc_body(f_ref, k_ref, idx_ref, conf_ref):
        return _tc_top2_impl(f_ref, k_ref, idx_ref, conf_ref, br)
    return _tc_body


def _tc_top2_impl(f_ref, k_ref, idx_ref, conf_ref, BR):
    f = f_ref[...]
    big = jnp.float32(jnp.inf)

    m1 = jnp.full((BR,), big)
    i1 = jnp.full((BR,), KSIZE, jnp.int32)
    m2 = jnp.full((BR,), big)
    for j in range(KSIZE // CH):
        kc = k_ref[j * CH:(j + 1) * CH, :]
        s = lax.dot_general(
            kc, f, (((1,), (1,)), ((), ())),
            preferred_element_type=jnp.float32,
            precision=lax.Precision.DEFAULT,
        )
        knc = jnp.sum(kc * kc, axis=1, keepdims=True)
        s = knc - 2.0 * s
        cm1 = jnp.min(s, axis=0)
        col = lax.broadcasted_iota(jnp.int32, (CH, BR), 0) + j * CH
        ci1 = jnp.min(jnp.where(s == cm1[None, :], col, KSIZE), axis=0)
        s2 = jnp.where(col == ci1[None, :], big, s)
        cm2 = jnp.min(s2, axis=0)
        i1 = jnp.where(cm1 < m1, ci1, i1)
        m2 = jnp.minimum(jnp.maximum(m1, cm1), jnp.minimum(m2, cm2))
        m1 = jnp.minimum(m1, cm1)

    fn = jnp.sum(f * f, axis=1)
    d1 = jnp.sqrt(jnp.maximum(m1 + fn, 1e-12))
    d2 = jnp.sqrt(jnp.maximum(m2 + fn, 1e-12))
    idx_ref[...] = i1
    conf = 1.0 - d1 / (d2 + 1e-8)
    conf_ref[...] = jnp.broadcast_to(conf[:, None], (BR, 16))


def _tc_top2(f, K, br=BR, interpret=False):
    nrow = f.shape[0]
    grid = nrow // br
    return pl.pallas_call(
        _make_tc_body(br),
        grid=(grid,),
        in_specs=[
            pl.BlockSpec((br, D), lambda i: (i, 0)),
            pl.BlockSpec((KSIZE, D), lambda i: (0, 0)),
        ],
        out_specs=[
            pl.BlockSpec((br,), lambda i: (i,)),
            pl.BlockSpec((br, 16), lambda i: (i, 0)),
        ],
        out_shape=[
            jax.ShapeDtypeStruct((nrow,), jnp.int32),
            jax.ShapeDtypeStruct((nrow, 16), jnp.float32),
        ],
        interpret=interpret,
    )(f, K)


NC, NS, L = 2, 16, 16
NW = NC * NS
RPW = NROW // NW
DCH = D // L


def _make_sc_body(rpw):
    def _sc_body(k_hbm, idx_hbm, conf_hbm, out_hbm, idx_v, conf_v, rows_v,
                 sem):
        wid = lax.axis_index("s") * NC + lax.axis_index("c")
        base = wid * rpw
        pltpu.sync_copy(idx_hbm.at[pl.ds(base, rpw)], idx_v)
        pltpu.sync_copy(conf_hbm.at[pl.ds(base, rpw)], conf_v)
        pltpu.async_copy(k_hbm.at[idx_v], rows_v, sem).wait()

        @plsc.parallel_loop(0, rpw, 1, unroll=2)
        def _(r):
            cv = conf_v[r, :]
            for d in range(DCH):
                sl = pl.ds(d * L, L)
                rows_v[r, sl] = rows_v[r, sl] * cv

        pltpu.sync_copy(rows_v, out_hbm.at[pl.ds(base, rpw)])
    return _sc_body


@functools.cache
def _sc_gather_scale(nrow):
    rpw = nrow // NW
    return pl.kernel(
        _make_sc_body(rpw),
        out_type=jax.ShapeDtypeStruct((nrow, D), jnp.float32),
        mesh=plsc.VectorSubcoreMesh(core_axis_name="c", subcore_axis_name="s"),
        scratch_types=[
            pltpu.VMEM((rpw,), jnp.int32),
            pltpu.VMEM((rpw, L), jnp.float32),
            pltpu.VMEM((rpw, D), jnp.float32),
            pltpu.SemaphoreType.DMA,
        ],
    )


def kernel(f_ipm, K):
    orig_shape = f_ipm.shape
    f = f_ipm.reshape(-1, orig_shape[-1])
    idx, conf = _tc_top2(f, K)
    out = _sc_gather_scale(NROW)(K, idx, conf)
    return out.reshape(orig_shape)


# speedup vs baseline: 1.3185x; 1.0257x over previous
import functools

import jax
import jax.numpy as jnp
from jax import lax
from jax.experimental import pallas as pl
from jax.experimental.pallas import tpu as pltpu
from jax.experimental.pallas import tpu_sc as plsc

B, T, D = 16, 576, 256
KSIZE = 1024
NROW = B * T


BR = 3072
CH = 1024


def _make_tc_body(br):
    def _tc_body(f_ref, k_ref, idx_ref, conf_ref):
        return _tc_top2_impl(f_ref, k_ref, idx_ref, conf_ref, br)
    return _tc_body


def _tc_top2_impl(f_ref, k_ref, idx_ref, conf_ref, BR):
    f = f_ref[...]
    big = jnp.float32(jnp.inf)

    m1 = jnp.full((BR,), big)
    i1 = jnp.full((BR,), KSIZE, jnp.int32)
    m2 = jnp.full((BR,), big)
    for j in range(KSIZE // CH):
        kc = k_ref[j * CH:(j + 1) * CH, :]
        kc2 = kc * -2.0
        s = lax.dot_general(
            kc2, f, (((1,), (1,)), ((), ())),
            preferred_element_type=jnp.float32,
            precision=lax.Precision.DEFAULT,
        )
        knc = jnp.sum(kc * kc, axis=1, keepdims=True)
        s = s + knc
        cm1 = jnp.min(s, axis=0)
        col = lax.broadcasted_iota(jnp.int32, (CH, BR), 0) + j * CH
        ci1 = jnp.min(jnp.where(s == cm1[None, :], col, KSIZE), axis=0)
        s2 = jnp.where(col == ci1[None, :], big, s)
        cm2 = jnp.min(s2, axis=0)
        i1 = jnp.where(cm1 < m1, ci1, i1)
        m2 = jnp.minimum(jnp.maximum(m1, cm1), jnp.minimum(m2, cm2))
        m1 = jnp.minimum(m1, cm1)

    fn = jnp.sum(f * f, axis=1)
    d1 = jnp.sqrt(jnp.maximum(m1 + fn, 1e-12))
    d2 = jnp.sqrt(jnp.maximum(m2 + fn, 1e-12))
    idx_ref[...] = i1
    conf = 1.0 - d1 / (d2 + 1e-8)
    conf_ref[...] = jnp.broadcast_to(conf[:, None], (BR, 16))


def _tc_top2(f, K, br=BR, interpret=False):
    nrow = f.shape[0]
    grid = nrow // br
    return pl.pallas_call(
        _make_tc_body(br),
        grid=(grid,),
        in_specs=[
            pl.BlockSpec((br, D), lambda i: (i, 0)),
            pl.BlockSpec((KSIZE, D), lambda i: (0, 0)),
        ],
        out_specs=[
            pl.BlockSpec((br,), lambda i: (i,)),
            pl.BlockSpec((br, 16), lambda i: (i, 0)),
        ],
        out_shape=[
            jax.ShapeDtypeStruct((nrow,), jnp.int32),
            jax.ShapeDtypeStruct((nrow, 16), jnp.float32),
        ],
        interpret=interpret,
    )(f, K)


NC, NS, L = 2, 16, 16
NW = NC * NS
RPW = NROW // NW
DCH = D // L


def _make_sc_body(rpw):
    def _sc_body(k_hbm, idx_hbm, conf_hbm, out_hbm, idx_v, conf_v, rows_v,
                 sem):
        wid = lax.axis_index("s") * NC + lax.axis_index("c")
        base = wid * rpw
        pltpu.sync_copy(idx_hbm.at[pl.ds(base, rpw)], idx_v)
        pltpu.sync_copy(conf_hbm.at[pl.ds(base, rpw)], conf_v)
        pltpu.async_copy(k_hbm.at[idx_v], rows_v, sem).wait()

        @plsc.parallel_loop(0, rpw, 1, unroll=2)
        def _(r):
            cv = conf_v[r, :]
            for d in range(DCH):
                sl = pl.ds(d * L, L)
                rows_v[r, sl] = rows_v[r, sl] * cv

        pltpu.sync_copy(rows_v, out_hbm.at[pl.ds(base, rpw)])
    return _sc_body


@functools.cache
def _sc_gather_scale(nrow):
    rpw = nrow // NW
    return pl.kernel(
        _make_sc_body(rpw),
        out_type=jax.ShapeDtypeStruct((nrow, D), jnp.float32),
        mesh=plsc.VectorSubcoreMesh(core_axis_name="c", subcore_axis_name="s"),
        scratch_types=[
            pltpu.VMEM((rpw,), jnp.int32),
            pltpu.VMEM((rpw, L), jnp.float32),
            pltpu.VMEM((rpw, D), jnp.float32),
            pltpu.SemaphoreType.DMA,
        ],
    )


def kernel(f_ipm, K):
    orig_shape = f_ipm.shape
    f = f_ipm.reshape(-1, orig_shape[-1])
    idx, conf = _tc_top2(f, K)
    out = _sc_gather_scale(NROW)(K, idx, conf)
    return out.reshape(orig_shape)
